# Initial kernel scaffold; baseline (speedup 1.0000x reference)
import jax
import jax.numpy as jnp
from jax.experimental import pallas as pl


def kernel(x, edge_index, W, att_src, att_dst, bias):
    raise NotImplementedError("write your pallas kernel here")


# speedup vs baseline: 8.8011x; 8.8011x over previous
import dataclasses
import functools

import jax
import jax.numpy as jnp
from jax import lax
from jax.experimental import pallas as pl
from jax.experimental.pallas import tpu as pltpu
from jax.experimental.pallas import tpu_sc as plsc

N = 10000
NP = 10240
NPR = NP // 128
D = 128
E_IN = 320000
E_REAL = E_IN + N
LANES = 128
NCORE = 2
NSUB = 16
ROWS_B = 88
ROWS_A = ROWS_B * NCORE
ROWS = NSUB * ROWS_A
EP = ROWS * LANES
NODE_SLAB = NP // NSUB
CH = 8
TC_BLK = 1024


def _prep_body(x_ref, w_ref, asv_ref, adv_ref, h_ref, as_ref, ad_ref):
    h = jnp.dot(x_ref[...], w_ref[...], preferred_element_type=jnp.float32)
    h_ref[...] = h
    as_ref[...] = jnp.sum(h * asv_ref[...], axis=1)
    ad_ref[...] = jnp.sum(h * adv_ref[...], axis=1)


_prep = pl.pallas_call(
    _prep_body,
    grid=(NP // TC_BLK,),
    in_specs=[
        pl.BlockSpec((TC_BLK, D), lambda i: (i, 0)),
        pl.BlockSpec((D, D), lambda i: (0, 0)),
        pl.BlockSpec((1, D), lambda i: (0, 0)),
        pl.BlockSpec((1, D), lambda i: (0, 0)),
    ],
    out_specs=[
        pl.BlockSpec((TC_BLK, D), lambda i: (i, 0)),
        pl.BlockSpec((TC_BLK,), lambda i: (i,)),
        pl.BlockSpec((TC_BLK,), lambda i: (i,)),
    ],
    out_shape=[
        jax.ShapeDtypeStruct((NP, D), jnp.float32),
        jax.ShapeDtypeStruct((NP,), jnp.float32),
        jax.ShapeDtypeStruct((NP,), jnp.float32),
    ],
)


def _fin_body(p0_ref, p1_ref, b_ref, o_ref):
    o_ref[...] = p0_ref[...] + p1_ref[...] + b_ref[...]


_finish = pl.pallas_call(
    _fin_body,
    grid=(NP // TC_BLK,),
    in_specs=[
        pl.BlockSpec((TC_BLK, D), lambda i: (i, 0)),
        pl.BlockSpec((TC_BLK, D), lambda i: (i, 0)),
        pl.BlockSpec((1, D), lambda i: (0, 0)),
    ],
    out_specs=pl.BlockSpec((TC_BLK, D), lambda i: (i, 0)),
    out_shape=jax.ShapeDtypeStruct((NP, D), jnp.float32),
)


def _sc_mesh_and_params():
    mesh = plsc.VectorSubcoreMesh(core_axis_name="c", subcore_axis_name="s")
    cp = pltpu.CompilerParams()
    if "needs_layout_passes" in pltpu.CompilerParams.__dataclass_fields__:
        cp = dataclasses.replace(cp, needs_layout_passes=False)
    return mesh, cp


def _sc_scores(src2d, dst2d, asrc, adst):
    mesh, cp = _sc_mesh_and_params()

    @functools.partial(
        pl.kernel,
        out_type=[
            jax.ShapeDtypeStruct((ROWS, LANES), jnp.float32),
            jax.ShapeDtypeStruct((NP,), jnp.float32),
            jax.ShapeDtypeStruct((NP,), jnp.float32),
        ],
        mesh=mesh,
        compiler_params=cp,
        scratch_types=[
            pltpu.VMEM((NP,), jnp.float32),
            pltpu.VMEM((NP,), jnp.float32),
            pltpu.VMEM((NP,), jnp.float32),
            pltpu.VMEM((ROWS_B, LANES), jnp.int32),
            pltpu.VMEM((ROWS_B, LANES), jnp.int32),
            pltpu.VMEM((ROWS_B, LANES), jnp.float32),
            pltpu.VMEM_SHARED((NP,), jnp.float32),
        ],
    )
    def body(src_hbm, dst_hbm, asrc_hbm, adst_hbm,
             e_hbm, denp0_hbm, denp1_hbm,
             asrc_v, adst_v, zero_v, src_v, dst_v, e_v, den_sh):
        c = lax.axis_index("c")
        s = lax.axis_index("s")
        zeros16 = jnp.zeros((16,), jnp.float32)

        pltpu.sync_copy(asrc_hbm, asrc_v)
        pltpu.sync_copy(adst_hbm, adst_v)

        @pl.when(s == 0)
        def _():
            @pl.loop(0, NP // 16)
            def _(i):
                zero_v[pl.ds(i * 16, 16)] = zeros16
            pltpu.sync_copy(zero_v, den_sh)

        plsc.subcore_barrier()

        row0 = s * ROWS_A + c * ROWS_B
        pltpu.sync_copy(src_hbm.at[pl.ds(row0, ROWS_B)], src_v)
        pltpu.sync_copy(dst_hbm.at[pl.ds(row0, ROWS_B)], dst_v)

        @pl.loop(0, ROWS_B)
        def _(j):
            gid0 = (row0 + j) * LANES
            for k in range(8):
                sl = pl.ds(k * 16, 16)
                s16 = src_v[j, sl]
                d16 = dst_v[j, sl]
                g = (plsc.load_gather(asrc_v, [s16])
                     + plsc.load_gather(adst_v, [d16]))
                a = jnp.where(g >= 0.0, g, g * jnp.float32(0.2))
                e = jnp.exp(a)
                gid = gid0 + k * 16 + lax.iota(jnp.int32, 16)
                e = jnp.where(gid < E_REAL, e, jnp.float32(0.0))
                e_v[j, sl] = e
            pltpu.sync_copy(e_v.at[j], den_sh.at[dst_v.at[j]], add=True)

        pltpu.sync_copy(e_v, e_hbm.at[pl.ds(row0, ROWS_B)])

        plsc.subcore_barrier()

        @pl.when((s == 0) & (c == 0))
        def _():
            pltpu.sync_copy(den_sh, denp0_hbm)

        @pl.when((s == 0) & (c == 1))
        def _():
            pltpu.sync_copy(den_sh, denp1_hbm)

    return body(src2d, dst2d, asrc, adst)


def _sc_aggregate(src2d, dst2d, e2d, denp0, denp1, h):
    mesh, cp = _sc_mesh_and_params()

    @functools.partial(
        pl.kernel,
        out_type=[
            jax.ShapeDtypeStruct((ROWS, LANES), jnp.float32),
            jax.ShapeDtypeStruct((NCORE, NP, D), jnp.float32),
        ],
        mesh=mesh,
        compiler_params=cp,
        scratch_types=[
            pltpu.VMEM((NP,), jnp.float32),
            pltpu.VMEM((NP,), jnp.float32),
            pltpu.VMEM((CH, LANES), jnp.int32),
            pltpu.VMEM((CH, LANES), jnp.int32),
            pltpu.VMEM((CH, LANES), jnp.float32),
            pltpu.VMEM((LANES, D), jnp.float32),
            pltpu.VMEM_SHARED((NP, D), jnp.float32),
            pltpu.SemaphoreType.DMA,
        ],
    )
    def body(src_hbm, dst_hbm, e_hbm, denp0_hbm, denp1_hbm, h_hbm,
             alpha_hbm, part_hbm,
             den_v, den2_v, src_v, dst_v, e_v, rows_v, acc_sh, sem):
        c = lax.axis_index("c")
        s = lax.axis_index("s")
        zeros16 = jnp.zeros((16,), jnp.float32)

        pltpu.sync_copy(denp0_hbm, den_v)
        pltpu.sync_copy(denp1_hbm, den2_v)

        @pl.loop(0, NP // 16)
        def _(i):
            sl = pl.ds(i * 16, 16)
            den_v[sl] = den_v[sl] + den2_v[sl]

        @pl.loop(0, LANES)
        def _(r):
            for k in range(8):
                rows_v[r, pl.ds(k * 16, 16)] = zeros16

        for i in range(NODE_SLAB // LANES):
            pltpu.sync_copy(rows_v,
                            acc_sh.at[pl.ds(s * NODE_SLAB + i * LANES, LANES)])

        plsc.subcore_barrier()

        @pl.loop(0, ROWS_B // CH)
        def _(cb):
            row_b = s * ROWS_A + c * ROWS_B + cb * CH
            pltpu.sync_copy(src_hbm.at[pl.ds(row_b, CH)], src_v)
            pltpu.sync_copy(dst_hbm.at[pl.ds(row_b, CH)], dst_v)
            pltpu.sync_copy(e_hbm.at[pl.ds(row_b, CH)], e_v)

            for r in range(CH):
                for k in range(8):
                    sl = pl.ds(k * 16, 16)
                    d16 = dst_v[r, sl]
                    den16 = plsc.load_gather(den_v, [d16])
                    e_v[r, sl] = e_v[r, sl] / (den16 + jnp.float32(1e-16))

            pltpu.sync_copy(e_v, alpha_hbm.at[pl.ds(row_b, CH)])

            for r in range(CH):
                copy = pltpu.async_copy(h_hbm.at[src_v.at[r]], rows_v, sem)
                copy.wait()

                @pl.loop(0, LANES)
                def _(rr, r=r):
                    av = plsc.load_gather(
                        e_v, [jnp.full((16,), r, jnp.int32),
                              jnp.full((16,), rr, jnp.int32)])
                    for k in range(8):
                        sl2 = (rr, pl.ds(k * 16, 16))
                        rows_v[sl2] = rows_v[sl2] * av

                pltpu.sync_copy(rows_v, acc_sh.at[dst_v.at[r]], add=True)

        plsc.subcore_barrier()

        pltpu.sync_copy(acc_sh.at[pl.ds(s * NODE_SLAB, NODE_SLAB)],
                        part_hbm.at[c, pl.ds(s * NODE_SLAB, NODE_SLAB)])

    return body(src2d, dst2d, e2d, denp0, denp1, h)


def kernel(x, edge_index, W, att_src, att_dst, bias):
    loop = jnp.arange(N, dtype=edge_index.dtype)
    ei = jnp.concatenate([edge_index, jnp.stack([loop, loop], axis=0)], axis=1)
    pad = jnp.zeros((2, EP - E_REAL), jnp.int32)
    eip = jnp.concatenate([ei, pad], axis=1)
    src2d = eip[0].reshape(ROWS, LANES)
    dst2d = eip[1].reshape(ROWS, LANES)
    xp = jnp.pad(x, ((0, NP - N), (0, 0)))
    h, asrc, adst = _prep(xp, W, att_src.reshape(1, D), att_dst.reshape(1, D))
    e2d, denp0, denp1 = _sc_scores(src2d, dst2d, asrc, adst)
    alpha2d, part = _sc_aggregate(src2d, dst2d, e2d, denp0, denp1, h)
    out = _finish(part[0], part[1], bias.reshape(1, D))
    alpha = alpha2d.reshape(-1)[:E_REAL]
    return out[:N], ei, alpha
